# Initial kernel scaffold; baseline (speedup 1.0000x reference)
import functools

import jax
import jax.numpy as jnp
from jax import lax
from jax.experimental import pallas as pl
from jax.experimental.pallas import tpu as pltpu
from jax.experimental.pallas import tpu_sc as plsc

_B = 16384
_D = 32
_H1, _H2 = 64, 32

_C = 64
_IDX_BATCH = 128


def _sc_gather_pool(user, movie, actor_flat, country_flat, type_flat,
                    user_tab, movie_tab, actor_tab, country_tab, type_tab):
  info = plsc.get_sparse_core_info()
  nw = info.num_cores * info.num_subcores
  bw = _B // nw
  nchunk = bw // _C

  mesh = plsc.VectorSubcoreMesh(core_axis_name="c", subcore_axis_name="s")

  out_t = jax.ShapeDtypeStruct((_B, _D), jnp.float32)

  @functools.partial(
      pl.kernel,
      mesh=mesh,
      out_type=[out_t] * 5,
      scratch_types=[
          pltpu.VMEM((_C,), jnp.int32),
          pltpu.VMEM((_C,), jnp.int32),
          pltpu.VMEM((_C * 20,), jnp.int32),
          pltpu.VMEM((_C * 4,), jnp.int32),
          pltpu.VMEM((_C * 4,), jnp.int32),
          pltpu.VMEM((_C, _D), jnp.float32),
          pltpu.VMEM((_C, _D), jnp.float32),
          pltpu.VMEM((_C * 20, _D), jnp.float32),
          pltpu.VMEM((_C * 4, _D), jnp.float32),
          pltpu.VMEM((_C * 4, _D), jnp.float32),
          pltpu.VMEM((_C, _D), jnp.float32),
          pltpu.VMEM((_C, _D), jnp.float32),
          pltpu.VMEM((_C, _D), jnp.float32),
          pltpu.SemaphoreType.DMA,
      ],
  )
  def body(user_i, movie_i, actor_i, country_i, type_i,
           ut, mt, at_, ct, tt,
           uo, mo, ao, co, to,
           idx_u, idx_m, idx_a, idx_c, idx_t,
           rows_u, rows_m, rows_a, rows_c, rows_t,
           pool_a, pool_c, pool_t, sem):
    wid = lax.axis_index("s") * info.num_cores + lax.axis_index("c")
    base = wid * bw

    def gather(tab, idx_ref, rows_ref, n):
      copies = []
      for off in range(0, n, _IDX_BATCH):
        sz = min(_IDX_BATCH, n - off)
        copies.append(pltpu.async_copy(
            tab.at[idx_ref.at[pl.ds(off, sz)]],
            rows_ref.at[pl.ds(off, sz)], sem))
      return copies

    def pool_loop(rows, pool, n, scale):
      def samp(c, _):
        r = c * n
        a0 = rows[r, pl.ds(0, 16)]
        a1 = rows[r, pl.ds(16, 16)]
        for j in range(1, n):
          a0 = a0 + rows[r + j, pl.ds(0, 16)]
          a1 = a1 + rows[r + j, pl.ds(16, 16)]
        pool[c, pl.ds(0, 16)] = a0 * scale
        pool[c, pl.ds(16, 16)] = a1 * scale
        return 0
      lax.fori_loop(0, _C, samp, 0)

    def chunk(k, _):
      cb = base + k * _C
      pltpu.sync_copy(user_i.at[pl.ds(cb, _C)], idx_u)
      pltpu.sync_copy(movie_i.at[pl.ds(cb, _C)], idx_m)
      pltpu.sync_copy(actor_i.at[pl.ds(cb * 20, _C * 20)], idx_a)
      pltpu.sync_copy(country_i.at[pl.ds(cb * 4, _C * 4)], idx_c)
      pltpu.sync_copy(type_i.at[pl.ds(cb * 4, _C * 4)], idx_t)

      cu = gather(ut, idx_u, rows_u, _C)
      cm = gather(mt, idx_m, rows_m, _C)
      ca = gather(at_, idx_a, rows_a, _C * 20)
      cc = gather(ct, idx_c, rows_c, _C * 4)
      ctp = gather(tt, idx_t, rows_t, _C * 4)

      for c in cu:
        c.wait()
      pltpu.sync_copy(rows_u, uo.at[pl.ds(cb, _C)])
      for c in cm:
        c.wait()
      pltpu.sync_copy(rows_m, mo.at[pl.ds(cb, _C)])
      for c in ca:
        c.wait()
      pool_loop(rows_a, pool_a, 20, 1.0 / 20.0)
      for c in cc:
        c.wait()
      pool_loop(rows_c, pool_c, 4, 0.25)
      for c in ctp:
        c.wait()
      pool_loop(rows_t, pool_t, 4, 0.25)

      pltpu.sync_copy(pool_a, ao.at[pl.ds(cb, _C)])
      pltpu.sync_copy(pool_c, co.at[pl.ds(cb, _C)])
      pltpu.sync_copy(pool_t, to.at[pl.ds(cb, _C)])
      return 0

    lax.fori_loop(0, nchunk, chunk, 0)

  return body(user, movie, actor_flat, country_flat, type_flat,
              user_tab, movie_tab, actor_tab, country_tab, type_tab)


_TB = 512


def _mlp_body(u, m, a, c, t, s, w1u, w1m, w1a, w1c, w1t, w1s, b1,
              w2, b2, w3, b3, o):
  hp = jax.lax.Precision.HIGHEST
  h = (jnp.dot(u[...], w1u[...], precision=hp)
       + jnp.dot(m[...], w1m[...], precision=hp)
       + jnp.dot(a[...], w1a[...], precision=hp)
       + jnp.dot(c[...], w1c[...], precision=hp)
       + jnp.dot(t[...], w1t[...], precision=hp)
       + jnp.dot(s[...], w1s[...], precision=hp)
       + b1[...])
  h = jnp.maximum(h, 0.0)
  h = jnp.maximum(jnp.dot(h, w2[...], precision=hp) + b2[...], 0.0)
  o[...] = jnp.dot(h, w3[...], precision=hp) + b3[...]


def _mlp(u, m, a, c, t, s, W1, b1, W2, b2, W3, b3):
  grid = _B // _TB
  feat_spec = pl.BlockSpec((_TB, _D), lambda i: (i, 0))
  s_spec = pl.BlockSpec((_TB, 4), lambda i: (i, 0))

  def full(shape):
    return pl.BlockSpec(shape, lambda i: tuple(0 for _ in shape))

  w1u, w1m, w1a, w1c, w1t = (W1[k * _D:(k + 1) * _D] for k in range(5))
  w1s = W1[5 * _D:]
  out = pl.pallas_call(
      _mlp_body,
      grid=(grid,),
      in_specs=[feat_spec] * 5 + [s_spec] + [
          full((_D, _H1))] * 5 + [full((4, _H1)), full((1, _H1)),
          full((_H1, _H2)), full((1, _H2)), full((_H2, 1)), full((1, 1))],
      out_specs=pl.BlockSpec((_TB, 1), lambda i: (i, 0)),
      out_shape=jax.ShapeDtypeStruct((_B, 1), jnp.float32),
  )(u, m, a, c, t, s, w1u, w1m, w1a, w1c, w1t, w1s, b1.reshape(1, _H1),
    W2, b2.reshape(1, _H2), W3, b3.reshape(1, 1))
  return out


@jax.jit
def kernel(user, movie, actor, country, movie_type, num_reviews,
           normalized_rating, useful_ratings, useless_ratings,
           user_table, movie_table, actor_table, country_table,
           movie_type_table, W1, b1, W2, b2, W3, b3):
  user = user.astype(jnp.int32)
  movie = movie.astype(jnp.int32)
  actor_flat = actor.reshape(-1).astype(jnp.int32)
  country_flat = country.reshape(-1).astype(jnp.int32)
  type_flat = movie_type.reshape(-1).astype(jnp.int32)

  u, m, a, c, t = _sc_gather_pool(
      user, movie, actor_flat, country_flat, type_flat,
      user_table, movie_table, actor_table, country_table,
      movie_type_table)

  s = jnp.stack([num_reviews, normalized_rating, useful_ratings,
                 useless_ratings], axis=1)
  out = _mlp(u, m, a, c, t, s, W1, b1, W2, b2, W3, b3)
  return jnp.squeeze(out, axis=-1)


# speedup vs baseline: 3.9346x; 3.9346x over previous
import functools

import jax
import jax.numpy as jnp
from jax import lax
from jax.experimental import pallas as pl
from jax.experimental.pallas import tpu as pltpu
from jax.experimental.pallas import tpu_sc as plsc

_B = 16384
_D = 32
_H1, _H2 = 64, 32

_C = 64
_IDX_BATCH = 128


def _sc_gather_pool(user, movie, actor_flat, country_flat, type_flat,
                    user_tab, movie_tab, actor_tab, country_tab, type_tab):
  info = plsc.get_sparse_core_info()
  nw = info.num_cores * info.num_subcores
  bw = _B // nw
  nchunk = bw // _C

  mesh = plsc.VectorSubcoreMesh(core_axis_name="c", subcore_axis_name="s")

  out_t = jax.ShapeDtypeStruct((_B, _D), jnp.float32)

  @functools.partial(
      pl.kernel,
      mesh=mesh,
      out_type=[out_t] * 5,
      compiler_params=pltpu.CompilerParams(use_tc_tiling_on_sc=False),
      scratch_types=[
          pltpu.VMEM((_C,), jnp.int32),
          pltpu.VMEM((_C,), jnp.int32),
          pltpu.VMEM((_C * 20,), jnp.int32),
          pltpu.VMEM((_C * 4,), jnp.int32),
          pltpu.VMEM((_C * 4,), jnp.int32),
          pltpu.VMEM((_C, _D), jnp.float32),
          pltpu.VMEM((_C, _D), jnp.float32),
          pltpu.VMEM((_C * 20, _D), jnp.float32),
          pltpu.VMEM((_C * 4, _D), jnp.float32),
          pltpu.VMEM((_C * 4, _D), jnp.float32),
          pltpu.VMEM((_C, _D), jnp.float32),
          pltpu.VMEM((_C, _D), jnp.float32),
          pltpu.VMEM((_C, _D), jnp.float32),
          pltpu.SemaphoreType.DMA,
      ],
  )
  def body(user_i, movie_i, actor_i, country_i, type_i,
           ut, mt, at_, ct, tt,
           uo, mo, ao, co, to,
           idx_u, idx_m, idx_a, idx_c, idx_t,
           rows_u, rows_m, rows_a, rows_c, rows_t,
           pool_a, pool_c, pool_t, sem):
    wid = lax.axis_index("s") * info.num_cores + lax.axis_index("c")
    base = wid * bw

    def gather(tab, idx_ref, rows_ref, n):
      copies = []
      for off in range(0, n, _IDX_BATCH):
        sz = min(_IDX_BATCH, n - off)
        copies.append(pltpu.async_copy(
            tab.at[idx_ref.at[pl.ds(off, sz)]],
            rows_ref.at[pl.ds(off, sz)], sem))
      return copies

    def pool_loop(rows, pool, n, scale):
      def samp(c, _):
        r = c * n
        a0 = rows[r, pl.ds(0, 16)]
        a1 = rows[r, pl.ds(16, 16)]
        for j in range(1, n):
          a0 = a0 + rows[r + j, pl.ds(0, 16)]
          a1 = a1 + rows[r + j, pl.ds(16, 16)]
        pool[c, pl.ds(0, 16)] = a0 * scale
        pool[c, pl.ds(16, 16)] = a1 * scale
        return 0
      lax.fori_loop(0, _C, samp, 0)

    def chunk(k, _):
      cb = base + k * _C
      pltpu.sync_copy(user_i.at[pl.ds(cb, _C)], idx_u)
      pltpu.sync_copy(movie_i.at[pl.ds(cb, _C)], idx_m)
      pltpu.sync_copy(actor_i.at[pl.ds(cb * 20, _C * 20)], idx_a)
      pltpu.sync_copy(country_i.at[pl.ds(cb * 4, _C * 4)], idx_c)
      pltpu.sync_copy(type_i.at[pl.ds(cb * 4, _C * 4)], idx_t)

      cu = gather(ut, idx_u, rows_u, _C)
      cm = gather(mt, idx_m, rows_m, _C)
      ca = gather(at_, idx_a, rows_a, _C * 20)
      cc = gather(ct, idx_c, rows_c, _C * 4)
      ctp = gather(tt, idx_t, rows_t, _C * 4)

      for c in cu:
        c.wait()
      pltpu.sync_copy(rows_u, uo.at[pl.ds(cb, _C)])
      for c in cm:
        c.wait()
      pltpu.sync_copy(rows_m, mo.at[pl.ds(cb, _C)])
      for c in ca:
        c.wait()
      pool_loop(rows_a, pool_a, 20, 1.0 / 20.0)
      for c in cc:
        c.wait()
      pool_loop(rows_c, pool_c, 4, 0.25)
      for c in ctp:
        c.wait()
      pool_loop(rows_t, pool_t, 4, 0.25)

      pltpu.sync_copy(pool_a, ao.at[pl.ds(cb, _C)])
      pltpu.sync_copy(pool_c, co.at[pl.ds(cb, _C)])
      pltpu.sync_copy(pool_t, to.at[pl.ds(cb, _C)])
      return 0

    lax.fori_loop(0, nchunk, chunk, 0)

  return body(user, movie, actor_flat, country_flat, type_flat,
              user_tab, movie_tab, actor_tab, country_tab, type_tab)


_TB = 512


def _mlp_body(u, m, a, c, t, s, w1u, w1m, w1a, w1c, w1t, w1s, b1,
              w2, b2, w3, b3, o):
  h = (jnp.dot(u[...], w1u[...])
       + jnp.dot(m[...], w1m[...])
       + jnp.dot(a[...], w1a[...])
       + jnp.dot(c[...], w1c[...])
       + jnp.dot(t[...], w1t[...])
       + jnp.dot(s[...], w1s[...])
       + b1[...])
  h = jnp.maximum(h, 0.0)
  h = jnp.maximum(jnp.dot(h, w2[...]) + b2[...], 0.0)
  o[...] = jnp.dot(h, w3[...]) + b3[...]


def _mlp(u, m, a, c, t, s, W1, b1, W2, b2, W3, b3):
  grid = _B // _TB
  feat_spec = pl.BlockSpec((_TB, _D), lambda i: (i, 0))
  s_spec = pl.BlockSpec((_TB, 4), lambda i: (i, 0))

  def full(shape):
    return pl.BlockSpec(shape, lambda i: tuple(0 for _ in shape))

  w1u, w1m, w1a, w1c, w1t = (W1[k * _D:(k + 1) * _D] for k in range(5))
  w1s = W1[5 * _D:]
  out = pl.pallas_call(
      _mlp_body,
      grid=(grid,),
      in_specs=[feat_spec] * 5 + [s_spec] + [
          full((_D, _H1))] * 5 + [full((4, _H1)), full((1, _H1)),
          full((_H1, _H2)), full((1, _H2)), full((_H2, 1)), full((1, 1))],
      out_specs=pl.BlockSpec((_TB, 1), lambda i: (i, 0)),
      out_shape=jax.ShapeDtypeStruct((_B, 1), jnp.float32),
  )(u, m, a, c, t, s, w1u, w1m, w1a, w1c, w1t, w1s, b1.reshape(1, _H1),
    W2, b2.reshape(1, _H2), W3, b3.reshape(1, 1))
  return out


@jax.jit
def kernel(user, movie, actor, country, movie_type, num_reviews,
           normalized_rating, useful_ratings, useless_ratings,
           user_table, movie_table, actor_table, country_table,
           movie_type_table, W1, b1, W2, b2, W3, b3):
  user = user.astype(jnp.int32)
  movie = movie.astype(jnp.int32)
  actor_flat = actor.reshape(-1).astype(jnp.int32)
  country_flat = country.reshape(-1).astype(jnp.int32)
  type_flat = movie_type.reshape(-1).astype(jnp.int32)

  u, m, a, c, t = _sc_gather_pool(
      user, movie, actor_flat, country_flat, type_flat,
      user_table, movie_table, actor_table, country_table,
      movie_type_table)

  s = jnp.stack([num_reviews, normalized_rating, useful_ratings,
                 useless_ratings], axis=1)
  out = _mlp(u, m, a, c, t, s, W1, b1, W2, b2, W3, b3)
  return jnp.squeeze(out, axis=-1)
